# Initial kernel scaffold; baseline (speedup 1.0000x reference)
import jax
import jax.numpy as jnp
from jax.experimental import pallas as pl


def kernel(x, edge_index, W1, b1, W2, b2):
    raise NotImplementedError("write your pallas kernel here")


# speedup vs baseline: 9.5791x; 9.5791x over previous
import functools

import jax
import jax.numpy as jnp
from jax import lax
from jax.experimental import pallas as pl
from jax.experimental.pallas import tpu as pltpu
from jax.experimental.pallas import tpu_sc as plsc

N = 10000
E = 320000
D = 128

NC = 2
NS = 16
NW = NC * NS
PAD_N = 10112
ROWS_PER_TILE = PAD_N // NS
CHUNK = 128
CH = 80
EPW = CH * CHUNK
PAD_E = NW * EPW

_mesh = plsc.VectorSubcoreMesh(core_axis_name="c", subcore_axis_name="s")


@functools.partial(
    pl.kernel,
    out_type=jax.ShapeDtypeStruct((NC, PAD_N, 128), jnp.float32),
    mesh=_mesh,
    scratch_types=[
        pltpu.VMEM((CH, CHUNK), jnp.int32),
        pltpu.VMEM((CHUNK, 128), jnp.float32),
        pltpu.VMEM_SHARED((PAD_N, 128), jnp.float32),
    ],
)
def _sc_degree(cols_hbm, ones_hbm, zdeg_hbm, out_hbm, colidx_v, ones_v, acc_sh):
    c = lax.axis_index("c")
    s = lax.axis_index("s")
    wid = c * NS + s
    pltpu.sync_copy(zdeg_hbm, acc_sh.at[pl.ds(s * ROWS_PER_TILE, ROWS_PER_TILE)])
    pltpu.sync_copy(ones_hbm, ones_v)
    pltpu.sync_copy(cols_hbm.at[pl.ds(wid * CH, CH)], colidx_v)
    plsc.subcore_barrier()

    def body(j, carry):
        pltpu.sync_copy(ones_v, acc_sh.at[colidx_v.at[j]], add=True)
        return carry

    lax.fori_loop(0, CH, body, 0)
    plsc.subcore_barrier()
    sl = pl.ds(s * ROWS_PER_TILE, ROWS_PER_TILE)
    pltpu.sync_copy(acc_sh.at[sl], out_hbm.at[c, sl])


@functools.partial(
    pl.kernel,
    out_type=jax.ShapeDtypeStruct((NC, PAD_N, D), jnp.float32),
    mesh=_mesh,
    scratch_types=[
        pltpu.VMEM((CH, CHUNK), jnp.int32),
        pltpu.VMEM((CH, CHUNK), jnp.int32),
        pltpu.VMEM((CHUNK, D), jnp.float32),
        pltpu.VMEM_SHARED((PAD_N, D), jnp.float32),
        pltpu.SemaphoreType.DMA,
    ],
)
def _sc_agg(y_hbm, rows_hbm, cols_hbm, zagg_hbm, out_hbm,
            rowidx_v, colidx_v, gbuf_v, acc_sh, sem):
    c = lax.axis_index("c")
    s = lax.axis_index("s")
    wid = c * NS + s
    pltpu.sync_copy(zagg_hbm, acc_sh.at[pl.ds(s * ROWS_PER_TILE, ROWS_PER_TILE)])
    pltpu.sync_copy(rows_hbm.at[pl.ds(wid * CH, CH)], rowidx_v)
    pltpu.sync_copy(cols_hbm.at[pl.ds(wid * CH, CH)], colidx_v)
    plsc.subcore_barrier()

    def body(j, carry):
        pltpu.async_copy(y_hbm.at[rowidx_v.at[j]], gbuf_v, sem).wait()
        pltpu.sync_copy(gbuf_v, acc_sh.at[colidx_v.at[j]], add=True)
        return carry

    lax.fori_loop(0, CH, body, 0)
    plsc.subcore_barrier()
    sl = pl.ds(s * ROWS_PER_TILE, ROWS_PER_TILE)
    pltpu.sync_copy(acc_sh.at[sl], out_hbm.at[c, sl])


def _dis(degp_ref):
    d = degp_ref[0, :, 0:1] + degp_ref[1, :, 0:1] + 1.0
    return lax.rsqrt(d)


def _tc_prep_body(degp_ref, x_ref, w_ref, y_ref):
    y_ref[...] = _dis(degp_ref) * jnp.dot(
        x_ref[...], w_ref[...], preferred_element_type=jnp.float32)


def _tc_mid_body(degp_ref, y1_ref, aggp_ref, b_ref, w_ref, y2_ref):
    dis = _dis(degp_ref)
    h = jnp.maximum(
        dis * (y1_ref[...] + aggp_ref[0] + aggp_ref[1]) + b_ref[...], 0.0)
    y2_ref[...] = dis * jnp.dot(h, w_ref[...],
                                preferred_element_type=jnp.float32)


def _tc_fin_body(degp_ref, y2_ref, aggp_ref, b_ref, out_ref):
    out_ref[...] = (_dis(degp_ref)
                    * (y2_ref[...] + aggp_ref[0] + aggp_ref[1]) + b_ref[...])


def _tc_call(body, out_shape, *args):
    return pl.pallas_call(
        body, out_shape=jax.ShapeDtypeStruct(out_shape, jnp.float32))(*args)


def kernel(x, edge_index, W1, b1, W2, b2):
    f32 = jnp.float32
    x_pad = jnp.pad(x, ((0, PAD_N - N), (0, 0)))
    rows = jnp.concatenate(
        [edge_index[0], jnp.zeros((PAD_E - E,), jnp.int32)]).reshape(NW * CH, CHUNK)
    cols = jnp.concatenate(
        [edge_index[1], jnp.full((PAD_E - E,), N, jnp.int32)]).reshape(NW * CH, CHUNK)
    ones16 = jnp.ones((CHUNK, 128), f32)
    zdeg = jnp.zeros((ROWS_PER_TILE, 128), f32)
    zagg = jnp.zeros((ROWS_PER_TILE, D), f32)
    b1r = b1.reshape(1, D)
    b2r = b2.reshape(1, D)

    degp = _sc_degree(cols, ones16, zdeg)
    y1 = _tc_call(_tc_prep_body, (PAD_N, D), degp, x_pad, W1)
    agg1 = _sc_agg(y1, rows, cols, zagg)
    y2 = _tc_call(_tc_mid_body, (PAD_N, D), degp, y1, agg1, b1r, W2)
    agg2 = _sc_agg(y2, rows, cols, zagg)
    out = _tc_call(_tc_fin_body, (PAD_N, D), degp, y2, agg2, b2r)
    return (out[:N], 0.0)
